# baseline (speedup 1.0000x reference)
import functools

import jax
import jax.numpy as jnp
import numpy as np
from jax import lax
from jax.experimental import pallas as pl
from jax.experimental.pallas import tpu as pltpu
from jax.experimental.pallas import tpu_sc as plsc

_SHAPES = np.array([[64, 64], [32, 32], [16, 16], [8, 8]], dtype=np.int64)
_LEVEL_START = np.array([0, 4096, 5120, 5376], dtype=np.int64)
_BS, _NQ, _NV, _D = 2, 10000, 5440, 256
_NH, _NL, _ASP, _NPNT = 8, 4, 8, 4
_DH = _D // _NH

_lanes = np.arange(_D)
_h = _lanes // (_NL * _ASP)
_l = (_lanes // _ASP) % _NL
_WL = _SHAPES[_l, 1].astype(np.float32)[None, :]
_HL = _SHAPES[_l, 0].astype(np.float32)[None, :]
_WLI = _SHAPES[_l, 1].astype(np.int32)[None, :]
_BASE = (_h * _NV + _LEVEL_START[_l]).astype(np.int32)[None, :]
_SEG = (( _lanes[:, None] // (_NL * _ASP)) == (_lanes[None, :] // (_NL * _ASP))).astype(np.float32)

_BQ = 1000
_BV = 680
_ITEMS = _BS * _NQ
_NW = 32
_PER_W = 640
_IB = 8


def _vproj_body(v_ref, w_ref, b_ref, o_ref):
    o_ref[0] = jnp.dot(v_ref[0], w_ref[...], preferred_element_type=jnp.float32) + b_ref[...]


def _prep_body(q_ref, rpx_ref, rpy_ref, wx_ref, bx_ref, wy_ref, by_ref,
               wa_ref, ba_ref, seg_ref, wl_ref, hl_ref, wli_ref, base_ref,
               idx_ref, w_ref):
    b = pl.program_id(0)
    q = q_ref[0]
    offx = jnp.dot(q, wx_ref[...], preferred_element_type=jnp.float32) + bx_ref[...]
    offy = jnp.dot(q, wy_ref[...], preferred_element_type=jnp.float32) + by_ref[...]
    a = jnp.dot(q, wa_ref[...], preferred_element_type=jnp.float32) + ba_ref[...]
    e = jnp.exp(a)
    ssum = jnp.dot(e, seg_ref[...], preferred_element_type=jnp.float32)
    aw = e / ssum

    wl = wl_ref[...]
    hl = hl_ref[...]
    wli = wli_ref[...]
    base = base_ref[...] + b * (_NH * _NV)

    x = rpx_ref[0] * wl + offx - 0.5
    y = rpy_ref[0] * hl + offy - 0.5
    x0 = jnp.floor(x)
    y0 = jnp.floor(y)

    xbf = jnp.clip(x0, 0.0, wl - 2.0)
    ybf = jnp.clip(y0, 0.0, hl - 2.0)
    xb = xbf.astype(jnp.int32)
    yb = ybf.astype(jnp.int32)
    wxl = jnp.maximum(0.0, 1.0 - jnp.abs(x - xbf))
    wxr = jnp.maximum(0.0, 1.0 - jnp.abs(x - xbf - 1.0))
    wy0 = jnp.maximum(0.0, 1.0 - jnp.abs(y - ybf))
    wy1 = jnp.maximum(0.0, 1.0 - jnp.abs(y - ybf - 1.0))
    idx_ref[0] = base + yb * wli + xb
    w_ref[0] = jnp.stack([wy0 * wxl * aw, wy0 * wxr * aw,
                          wy1 * wxl * aw, wy1 * wxr * aw], axis=1)


def _sc_body(table, idxr, wr, outr, idx_v, w_v, rows_v, out_v, sem_m, sem_g, sem_o):
    wid = lax.axis_index("s") * 2 + lax.axis_index("c")
    base = wid * _PER_W
    nreal = jnp.minimum(_PER_W // _IB, (_ITEMS - base) // _IB)

    def fire_meta(mb, blk):
        it0 = base + blk * _IB
        pltpu.async_copy(idxr.at[pl.ds(it0, _IB)], idx_v.at[mb], sem_m)
        pltpu.async_copy(wr.at[pl.ds(it0, _IB)], w_v.at[mb], sem_m)

    def wait_meta(mb, blk):
        it0 = base + blk * _IB
        pltpu.make_async_copy(idxr.at[pl.ds(it0, _IB)], idx_v.at[mb], sem_m).wait()
        pltpu.make_async_copy(wr.at[pl.ds(it0, _IB)], w_v.at[mb], sem_m).wait()

    def fire_item(mb, t):
        for k in range(2):
            pltpu.async_copy(table.at[idx_v.at[mb, t, k]],
                             rows_v.at[(2 * t) % 8 + k], sem_g)

    def wait_h(buf):
        pltpu.make_async_copy(table.at[pl.ds(0, 128)], rows_v.at[buf], sem_g).wait()

    iota2 = jnp.arange(16, dtype=jnp.int32) * 2

    def compute_half(mb, i, half, buf, ob):
        ob16 = jnp.full((16,), ob, jnp.int32)
        i16 = jnp.full((16,), i, jnp.int32)

        def h_body(h, _):
            aE = jnp.zeros((16,), jnp.float32)
            aO = jnp.zeros((16,), jnp.float32)
            hb = half * 128 + h * 32
            lb = h * 32
            for g in range(2):
                w00 = w_v[mb, i, 0, pl.ds(hb + g * 16, 16)]
                w01 = w_v[mb, i, 1, pl.ds(hb + g * 16, 16)]
                w10 = w_v[mb, i, 2, pl.ds(hb + g * 16, 16)]
                w11 = w_v[mb, i, 3, pl.ds(hb + g * 16, 16)]
                for j in range(16):
                    n = lb + g * 16 + j
                    r0 = plsc.bitcast(rows_v[buf, n, 0, pl.ds(0, 16)], jnp.bfloat16)
                    r1 = plsc.bitcast(rows_v[buf, n, 1, pl.ds(0, 16)], jnp.bfloat16)
                    r2 = plsc.bitcast(rows_v[buf, n, 2, pl.ds(0, 16)], jnp.bfloat16)
                    r3 = plsc.bitcast(rows_v[buf, n, 3, pl.ds(0, 16)], jnp.bfloat16)
                    b00 = plsc.bitcast(jnp.full((16,), w00[j]), jnp.bfloat16)
                    b01 = plsc.bitcast(jnp.full((16,), w01[j]), jnp.bfloat16)
                    b10 = plsc.bitcast(jnp.full((16,), w10[j]), jnp.bfloat16)
                    b11 = plsc.bitcast(jnp.full((16,), w11[j]), jnp.bfloat16)
                    acc = b00 * r0 + b01 * r1 + b10 * r2 + b11 * r3
                    e, o = plsc.unpack(acc, format=plsc.PackFormat.INTERLEAVED)
                    aE = aE + e
                    aO = aO + o
            plsc.store_scatter(out_v, [ob16, i16, hb + iota2], aE)
            plsc.store_scatter(out_v, [ob16, i16, hb + 1 + iota2], aO)
            return 0

        lax.fori_loop(0, _NH // 2, h_body, 0)

    def do_block(blk, mb):
        ob = mb
        it0 = base + blk * _IB
        has_next = blk + 1 < nreal

        @pl.when(has_next)
        def _():
            fire_meta(1 - mb, blk + 1)

        @pl.when(blk >= 2)
        def _():
            pltpu.make_async_copy(out_v.at[ob], outr.at[pl.ds(it0, _IB)], sem_o).wait()

        for i in range(_IB):
            b0 = (2 * i) % 8
            wait_h(b0)
            compute_half(mb, i, 0, b0, ob)
            wait_h(b0 + 1)
            compute_half(mb, i, 1, b0 + 1, ob)
            tgt = i + 4
            if tgt < _IB:
                fire_item(mb, tgt)
            elif tgt == _IB:
                @pl.when(has_next)
                def _():
                    wait_meta(1 - mb, blk + 1)
                    fire_item(1 - mb, 0)
            else:
                @pl.when(has_next)
                def _(tgt=tgt):
                    fire_item(1 - mb, tgt - _IB)
        pltpu.async_copy(out_v.at[ob], outr.at[pl.ds(it0, _IB)], sem_o)

    fire_meta(0, 0)
    wait_meta(0, 0)
    fire_item(0, 0)
    fire_item(0, 1)
    fire_item(0, 2)
    fire_item(0, 3)

    def pair_body(p, _):
        do_block(2 * p, 0)
        do_block(2 * p + 1, 1)
        return 0

    lax.fori_loop(0, nreal // 2, pair_body, 0)

    pltpu.make_async_copy(out_v.at[0], outr.at[pl.ds(base, _IB)], sem_o).wait()
    pltpu.make_async_copy(out_v.at[1], outr.at[pl.ds(base, _IB)], sem_o).wait()


def kernel(query, key, value, reference_points, spatial_shapes, level_start_index,
           W_value, b_value, W_off, b_off, W_attn, b_attn):
    bs, nq, dims = query.shape
    nv = value.shape[1]

    v2d = pl.pallas_call(
        _vproj_body,
        grid=(bs, nv // _BV),
        in_specs=[
            pl.BlockSpec((1, _BV, _D), lambda b, i: (b, i, 0)),
            pl.BlockSpec((_D, _D), lambda b, i: (0, 0)),
            pl.BlockSpec((1, _D), lambda b, i: (0, 0)),
        ],
        out_specs=pl.BlockSpec((1, _BV, _D), lambda b, i: (b, i, 0)),
        out_shape=jax.ShapeDtypeStruct((bs, nv, _D), jnp.float32),
    )(value, W_value.reshape(1, _D, _D)[0], b_value.reshape(1, _D))
    table = v2d.reshape(bs, nv, _NH, _DH).transpose(0, 2, 1, 3).reshape(bs * _NH * nv, _DH)
    tb = table.astype(jnp.bfloat16)
    col1 = jnp.roll(tb, -1, axis=0)
    tbv = tb.reshape(bs * _NH, nv, _DH)
    segs = []
    for l in range(_NL):
        s0, hw, wl_ = int(_LEVEL_START[l]), int(_SHAPES[l, 0] * _SHAPES[l, 1]), int(_SHAPES[l, 1])
        seg = tbv[:, s0 + wl_: s0 + hw]
        segs.append(jnp.pad(seg, ((0, 0), (0, wl_), (0, 0))))
    col2 = jnp.concatenate(segs, axis=1).reshape(bs * _NH * nv, _DH)
    col3 = jnp.roll(col2, -1, axis=0)
    quad = jnp.stack([tb, col1, col2, col3], axis=1)
    table_pairs = jax.lax.bitcast_convert_type(
        quad.reshape(bs * _NH * nv, 4, _DH // 2, 2), jnp.float32)

    w_off_r = W_off.reshape(_D, _NH * _NL * _ASP, 2)
    wx, wy = w_off_r[:, :, 0], w_off_r[:, :, 1]
    b_off_r = b_off.reshape(1, _NH * _NL * _ASP, 2)
    bx, by = b_off_r[:, :, 0], b_off_r[:, :, 1]
    rpx = jnp.tile(reference_points[..., 0], (1, 1, _D // _NPNT))
    rpy = jnp.tile(reference_points[..., 1], (1, 1, _D // _NPNT))

    idx, w = pl.pallas_call(
        _prep_body,
        grid=(bs, nq // _BQ),
        in_specs=[
            pl.BlockSpec((1, _BQ, _D), lambda b, i: (b, i, 0)),
            pl.BlockSpec((1, _BQ, _D), lambda b, i: (b, i, 0)),
            pl.BlockSpec((1, _BQ, _D), lambda b, i: (b, i, 0)),
            pl.BlockSpec((_D, _D), lambda b, i: (0, 0)),
            pl.BlockSpec((1, _D), lambda b, i: (0, 0)),
            pl.BlockSpec((_D, _D), lambda b, i: (0, 0)),
            pl.BlockSpec((1, _D), lambda b, i: (0, 0)),
            pl.BlockSpec((_D, _D), lambda b, i: (0, 0)),
            pl.BlockSpec((1, _D), lambda b, i: (0, 0)),
            pl.BlockSpec((_D, _D), lambda b, i: (0, 0)),
            pl.BlockSpec((1, _D), lambda b, i: (0, 0)),
            pl.BlockSpec((1, _D), lambda b, i: (0, 0)),
            pl.BlockSpec((1, _D), lambda b, i: (0, 0)),
            pl.BlockSpec((1, _D), lambda b, i: (0, 0)),
        ],
        out_specs=[
            pl.BlockSpec((1, _BQ, _D), lambda b, i: (b, i, 0)),
            pl.BlockSpec((1, _BQ, 4, _D), lambda b, i: (b, i, 0, 0)),
        ],
        out_shape=[
            jax.ShapeDtypeStruct((bs, nq, _D), jnp.int32),
            jax.ShapeDtypeStruct((bs, nq, 4, _D), jnp.float32),
        ],
    )(query, rpx, rpy, wx, bx, wy, by, W_attn, b_attn.reshape(1, _D),
      jnp.asarray(_SEG), jnp.asarray(_WL), jnp.asarray(_HL),
      jnp.asarray(_WLI), jnp.asarray(_BASE))

    idx_sc = idx.reshape(_ITEMS, 2, 128)
    wb = w.astype(jnp.bfloat16)
    w_pk = jax.lax.bitcast_convert_type(jnp.stack([wb, wb], axis=-1), jnp.float32)
    w_sc = w_pk.reshape(_ITEMS, 4, _D)

    mesh = plsc.VectorSubcoreMesh(core_axis_name="c", subcore_axis_name="s",
                                  num_cores=2, num_subcores=16)
    sc = pl.kernel(
        _sc_body,
        out_type=jax.ShapeDtypeStruct((_ITEMS, _D), jnp.float32),
        mesh=mesh,
        compiler_params=pltpu.CompilerParams(use_tc_tiling_on_sc=False,
                                             needs_layout_passes=False),
        scratch_types=[
            pltpu.VMEM((2, _IB, 2, 128), jnp.int32),
            pltpu.VMEM((2, _IB, 4, _D), jnp.float32),
            pltpu.VMEM((8, 128, 4, _DH // 2), jnp.float32),
            pltpu.VMEM((2, _IB, _D), jnp.float32),
            pltpu.SemaphoreType.DMA,
            pltpu.SemaphoreType.DMA,
            pltpu.SemaphoreType.DMA,
        ],
    )
    out = sc(table_pairs, idx_sc, w_sc)
    return out.reshape(bs, nq, _D)


# speedup vs baseline: 1.0168x; 1.0168x over previous
import functools

import jax
import jax.numpy as jnp
import numpy as np
from jax import lax
from jax.experimental import pallas as pl
from jax.experimental.pallas import tpu as pltpu
from jax.experimental.pallas import tpu_sc as plsc

_SHAPES = np.array([[64, 64], [32, 32], [16, 16], [8, 8]], dtype=np.int64)
_LEVEL_START = np.array([0, 4096, 5120, 5376], dtype=np.int64)
_BS, _NQ, _NV, _D = 2, 10000, 5440, 256
_NH, _NL, _ASP, _NPNT = 8, 4, 8, 4
_DH = _D // _NH

_lanes = np.arange(_D)
_h = _lanes // (_NL * _ASP)
_l = (_lanes // _ASP) % _NL
_WL = _SHAPES[_l, 1].astype(np.float32)[None, :]
_HL = _SHAPES[_l, 0].astype(np.float32)[None, :]
_WLI = _SHAPES[_l, 1].astype(np.int32)[None, :]
_BASE = (_h * _NV + _LEVEL_START[_l]).astype(np.int32)[None, :]
_SEG = (( _lanes[:, None] // (_NL * _ASP)) == (_lanes[None, :] // (_NL * _ASP))).astype(np.float32)

_BQ = 1000
_BV = 680
_ITEMS = _BS * _NQ
_NW = 32
_PER_W = 640
_IB = 8


def _vproj_body(v_ref, w_ref, b_ref, o_ref):
    o_ref[0] = jnp.dot(v_ref[0], w_ref[...], preferred_element_type=jnp.float32) + b_ref[...]


def _prep_body(q_ref, rpx_ref, rpy_ref, wx_ref, bx_ref, wy_ref, by_ref,
               wa_ref, ba_ref, seg_ref, wl_ref, hl_ref, wli_ref, base_ref,
               idx_ref, w_ref):
    b = pl.program_id(0)
    q = q_ref[0]
    offx = jnp.dot(q, wx_ref[...], preferred_element_type=jnp.float32) + bx_ref[...]
    offy = jnp.dot(q, wy_ref[...], preferred_element_type=jnp.float32) + by_ref[...]
    a = jnp.dot(q, wa_ref[...], preferred_element_type=jnp.float32) + ba_ref[...]
    e = jnp.exp(a)
    ssum = jnp.dot(e, seg_ref[...], preferred_element_type=jnp.float32)
    aw = e / ssum

    wl = wl_ref[...]
    hl = hl_ref[...]
    wli = wli_ref[...]
    base = base_ref[...] + b * (_NH * _NV)

    x = rpx_ref[0] * wl + offx - 0.5
    y = rpy_ref[0] * hl + offy - 0.5
    x0 = jnp.floor(x)
    y0 = jnp.floor(y)

    xbf = jnp.clip(x0, 0.0, wl - 2.0)
    ybf = jnp.clip(y0, 0.0, hl - 2.0)
    xb = xbf.astype(jnp.int32)
    yb = ybf.astype(jnp.int32)
    wxl = jnp.maximum(0.0, 1.0 - jnp.abs(x - xbf))
    wxr = jnp.maximum(0.0, 1.0 - jnp.abs(x - xbf - 1.0))
    wy0 = jnp.maximum(0.0, 1.0 - jnp.abs(y - ybf))
    wy1 = jnp.maximum(0.0, 1.0 - jnp.abs(y - ybf - 1.0))
    idx_ref[0] = base + yb * wli + xb
    w_ref[0] = jnp.stack([wy0 * wxl * aw, wy0 * wxr * aw,
                          wy1 * wxl * aw, wy1 * wxr * aw], axis=1)


def _sc_body(table, idxr, wr, outr, idx_v, w_v, rows_v, out_v, sem_m, sem_g, sem_o):
    wid = lax.axis_index("s") * 2 + lax.axis_index("c")
    base = wid * _PER_W
    nreal = jnp.minimum(_PER_W // _IB, (_ITEMS - base) // _IB)

    def fire_meta(mb, blk):
        it0 = base + blk * _IB
        pltpu.async_copy(idxr.at[pl.ds(it0, _IB)], idx_v.at[mb], sem_m)
        pltpu.async_copy(wr.at[pl.ds(it0, _IB)], w_v.at[mb], sem_m)

    def wait_meta(mb, blk):
        it0 = base + blk * _IB
        pltpu.make_async_copy(idxr.at[pl.ds(it0, _IB)], idx_v.at[mb], sem_m).wait()
        pltpu.make_async_copy(wr.at[pl.ds(it0, _IB)], w_v.at[mb], sem_m).wait()

    def fire_g(mb, i, rb):
        for k in range(2):
            pltpu.async_copy(table.at[idx_v.at[mb, i, k]],
                             rows_v.at[rb, pl.ds(k * 128, 128)], sem_g)

    def wait_g(rb):
        pltpu.make_async_copy(table.at[pl.ds(0, 256)], rows_v.at[rb], sem_g).wait()

    iota2 = jnp.arange(16, dtype=jnp.int32) * 2

    def compute(mb, i, rb, ob):
        ob16 = jnp.full((16,), ob, jnp.int32)
        i16 = jnp.full((16,), i, jnp.int32)

        def h_body(h, _):
            aE = jnp.zeros((16,), jnp.float32)
            aO = jnp.zeros((16,), jnp.float32)
            hb = h * 32
            lb = h * 32
            for g in range(2):
                w00 = w_v[mb, i, 0, pl.ds(hb + g * 16, 16)]
                w01 = w_v[mb, i, 1, pl.ds(hb + g * 16, 16)]
                w10 = w_v[mb, i, 2, pl.ds(hb + g * 16, 16)]
                w11 = w_v[mb, i, 3, pl.ds(hb + g * 16, 16)]
                for j in range(16):
                    n = lb + g * 16 + j
                    r0 = plsc.bitcast(rows_v[rb, n, 0, pl.ds(0, 16)], jnp.bfloat16)
                    r1 = plsc.bitcast(rows_v[rb, n, 1, pl.ds(0, 16)], jnp.bfloat16)
                    r2 = plsc.bitcast(rows_v[rb, n, 2, pl.ds(0, 16)], jnp.bfloat16)
                    r3 = plsc.bitcast(rows_v[rb, n, 3, pl.ds(0, 16)], jnp.bfloat16)
                    b00 = plsc.bitcast(jnp.full((16,), w00[j]), jnp.bfloat16)
                    b01 = plsc.bitcast(jnp.full((16,), w01[j]), jnp.bfloat16)
                    b10 = plsc.bitcast(jnp.full((16,), w10[j]), jnp.bfloat16)
                    b11 = plsc.bitcast(jnp.full((16,), w11[j]), jnp.bfloat16)
                    acc = b00 * r0 + b01 * r1 + b10 * r2 + b11 * r3
                    e, o = plsc.unpack(acc, format=plsc.PackFormat.INTERLEAVED)
                    aE = aE + e
                    aO = aO + o
            plsc.store_scatter(out_v, [ob16, i16, hb + iota2], aE)
            plsc.store_scatter(out_v, [ob16, i16, hb + 1 + iota2], aO)
            return 0

        lax.fori_loop(0, _NH, h_body, 0)

    def do_block(blk, mb):
        ob = mb
        it0 = base + blk * _IB
        has_next = blk + 1 < nreal

        @pl.when(has_next)
        def _():
            fire_meta(1 - mb, blk + 1)

        @pl.when(blk >= 2)
        def _():
            pltpu.make_async_copy(out_v.at[ob], outr.at[pl.ds(it0, _IB)], sem_o).wait()

        for i in range(_IB):
            wait_g(i % 4)
            nxt = i + 3
            if nxt < _IB:
                fire_g(mb, nxt, nxt % 4)
            elif nxt == _IB:
                @pl.when(has_next)
                def _():
                    wait_meta(1 - mb, blk + 1)
                    fire_g(1 - mb, 0, nxt % 4)
            else:
                @pl.when(has_next)
                def _(nxt=nxt):
                    fire_g(1 - mb, nxt - _IB, nxt % 4)
            compute(mb, i, i % 4, ob)
        pltpu.async_copy(out_v.at[ob], outr.at[pl.ds(it0, _IB)], sem_o)

    fire_meta(0, 0)
    wait_meta(0, 0)
    fire_g(0, 0, 0)
    fire_g(0, 1, 1)
    fire_g(0, 2, 2)

    def pair_body(p, _):
        do_block(2 * p, 0)
        do_block(2 * p + 1, 1)
        return 0

    lax.fori_loop(0, nreal // 2, pair_body, 0)

    pltpu.make_async_copy(out_v.at[0], outr.at[pl.ds(base, _IB)], sem_o).wait()
    pltpu.make_async_copy(out_v.at[1], outr.at[pl.ds(base, _IB)], sem_o).wait()


def kernel(query, key, value, reference_points, spatial_shapes, level_start_index,
           W_value, b_value, W_off, b_off, W_attn, b_attn):
    bs, nq, dims = query.shape
    nv = value.shape[1]

    v2d = pl.pallas_call(
        _vproj_body,
        grid=(bs, nv // _BV),
        in_specs=[
            pl.BlockSpec((1, _BV, _D), lambda b, i: (b, i, 0)),
            pl.BlockSpec((_D, _D), lambda b, i: (0, 0)),
            pl.BlockSpec((1, _D), lambda b, i: (0, 0)),
        ],
        out_specs=pl.BlockSpec((1, _BV, _D), lambda b, i: (b, i, 0)),
        out_shape=jax.ShapeDtypeStruct((bs, nv, _D), jnp.float32),
    )(value, W_value.reshape(1, _D, _D)[0], b_value.reshape(1, _D))
    table = v2d.reshape(bs, nv, _NH, _DH).transpose(0, 2, 1, 3).reshape(bs * _NH * nv, _DH)
    tb = table.astype(jnp.bfloat16)
    col1 = jnp.roll(tb, -1, axis=0)
    tbv = tb.reshape(bs * _NH, nv, _DH)
    segs = []
    for l in range(_NL):
        s0, hw, wl_ = int(_LEVEL_START[l]), int(_SHAPES[l, 0] * _SHAPES[l, 1]), int(_SHAPES[l, 1])
        seg = tbv[:, s0 + wl_: s0 + hw]
        segs.append(jnp.pad(seg, ((0, 0), (0, wl_), (0, 0))))
    col2 = jnp.concatenate(segs, axis=1).reshape(bs * _NH * nv, _DH)
    col3 = jnp.roll(col2, -1, axis=0)
    quad = jnp.stack([tb, col1, col2, col3], axis=1)
    table_pairs = jax.lax.bitcast_convert_type(
        quad.reshape(bs * _NH * nv, 4, _DH // 2, 2), jnp.float32)

    w_off_r = W_off.reshape(_D, _NH * _NL * _ASP, 2)
    wx, wy = w_off_r[:, :, 0], w_off_r[:, :, 1]
    b_off_r = b_off.reshape(1, _NH * _NL * _ASP, 2)
    bx, by = b_off_r[:, :, 0], b_off_r[:, :, 1]
    rpx = jnp.tile(reference_points[..., 0], (1, 1, _D // _NPNT))
    rpy = jnp.tile(reference_points[..., 1], (1, 1, _D // _NPNT))

    idx, w = pl.pallas_call(
        _prep_body,
        grid=(bs, nq // _BQ),
        in_specs=[
            pl.BlockSpec((1, _BQ, _D), lambda b, i: (b, i, 0)),
            pl.BlockSpec((1, _BQ, _D), lambda b, i: (b, i, 0)),
            pl.BlockSpec((1, _BQ, _D), lambda b, i: (b, i, 0)),
            pl.BlockSpec((_D, _D), lambda b, i: (0, 0)),
            pl.BlockSpec((1, _D), lambda b, i: (0, 0)),
            pl.BlockSpec((_D, _D), lambda b, i: (0, 0)),
            pl.BlockSpec((1, _D), lambda b, i: (0, 0)),
            pl.BlockSpec((_D, _D), lambda b, i: (0, 0)),
            pl.BlockSpec((1, _D), lambda b, i: (0, 0)),
            pl.BlockSpec((_D, _D), lambda b, i: (0, 0)),
            pl.BlockSpec((1, _D), lambda b, i: (0, 0)),
            pl.BlockSpec((1, _D), lambda b, i: (0, 0)),
            pl.BlockSpec((1, _D), lambda b, i: (0, 0)),
            pl.BlockSpec((1, _D), lambda b, i: (0, 0)),
        ],
        out_specs=[
            pl.BlockSpec((1, _BQ, _D), lambda b, i: (b, i, 0)),
            pl.BlockSpec((1, _BQ, 4, _D), lambda b, i: (b, i, 0, 0)),
        ],
        out_shape=[
            jax.ShapeDtypeStruct((bs, nq, _D), jnp.int32),
            jax.ShapeDtypeStruct((bs, nq, 4, _D), jnp.float32),
        ],
    )(query, rpx, rpy, wx, bx, wy, by, W_attn, b_attn.reshape(1, _D),
      jnp.asarray(_SEG), jnp.asarray(_WL), jnp.asarray(_HL),
      jnp.asarray(_WLI), jnp.asarray(_BASE))

    idx_sc = idx.reshape(_ITEMS, 2, 128)
    wb = w.astype(jnp.bfloat16)
    w_pk = jax.lax.bitcast_convert_type(jnp.stack([wb, wb], axis=-1), jnp.float32)
    w_sc = w_pk.reshape(_ITEMS, 4, _D)

    mesh = plsc.VectorSubcoreMesh(core_axis_name="c", subcore_axis_name="s",
                                  num_cores=2, num_subcores=16)
    sc = pl.kernel(
        _sc_body,
        out_type=jax.ShapeDtypeStruct((_ITEMS, _D), jnp.float32),
        mesh=mesh,
        compiler_params=pltpu.CompilerParams(use_tc_tiling_on_sc=False,
                                             needs_layout_passes=False),
        scratch_types=[
            pltpu.VMEM((2, _IB, 2, 128), jnp.int32),
            pltpu.VMEM((2, _IB, 4, _D), jnp.float32),
            pltpu.VMEM((4, _D, 4, _DH // 2), jnp.float32),
            pltpu.VMEM((2, _IB, _D), jnp.float32),
            pltpu.SemaphoreType.DMA,
            pltpu.SemaphoreType.DMA,
            pltpu.SemaphoreType.DMA,
        ],
    )
    out = sc(table_pairs, idx_sc, w_sc)
    return out.reshape(bs, nq, _D)
